# baseline (speedup 1.0000x reference)
import functools

import jax
import jax.numpy as jnp
from jax import lax
from jax.experimental import pallas as pl
from jax.experimental.pallas import tpu as pltpu
from jax.experimental.pallas import tpu_sc as plsc

N = 10000
E = 320000
D = 128
G = 16
DOUT = 64

NC = 2
NS = 16
L = 16
NW = NC * NS

CH = 128
NCHUNK = 79
N0 = 128
N1 = 30
EPT = NCHUNK * CH
EPAD = EPT * NW
NROWPAD = 10112
RPT = NROWPAD // NS
HR = EPT // L
DEGR = 80
NHIST = DEGR * D

_sc_mesh = plsc.VectorSubcoreMesh(core_axis_name="c", subcore_axis_name="s")
_sc_params = pltpu.CompilerParams(needs_layout_passes=False)


@functools.partial(
    pl.kernel,
    out_type=jax.ShapeDtypeStruct((NW * NHIST,), jnp.float32),
    mesh=_sc_mesh,
    compiler_params=_sc_params,
    scratch_types=[
        pltpu.VMEM((EPT,), jnp.int32),
        pltpu.VMEM((NHIST,), jnp.float32),
    ],
)
def _sc_degree(dst_hbm, zero_hbm, deg_hbm, didx, hist):
    c = lax.axis_index("c")
    s = lax.axis_index("s")
    wid = s * NC + c
    base = pl.multiple_of(wid * EPT, CH)
    pltpu.sync_copy(dst_hbm.at[pl.ds(base, EPT)], didx)
    pltpu.sync_copy(zero_hbm, hist)
    ones = jnp.ones((L,), jnp.float32)

    def step(j, carry):
        idx = didx[pl.ds(j * L, L)]
        plsc.addupdate_scatter(hist, [idx], ones)
        return carry

    lax.fori_loop(0, HR, step, 0)
    pltpu.sync_copy(hist, deg_hbm.at[pl.ds(pl.multiple_of(wid * NHIST, CH), NHIST)])


K = 3


@functools.partial(
    pl.kernel,
    out_type=jax.ShapeDtypeStruct((NC, NROWPAD, D), jnp.float32),
    mesh=_sc_mesh,
    compiler_params=_sc_params,
    scratch_types=[
        pltpu.VMEM((K, CH), jnp.int32),
        pltpu.VMEM((K, CH), jnp.int32),
    ] + [pltpu.VMEM((CH, D), jnp.float32) for _ in range(K)] + [
        pltpu.VMEM_SHARED((NROWPAD, D), jnp.float32),
    ] + [pltpu.SemaphoreType.DMA for _ in range(K)],
)
def _sc_scatter(hs_hbm, src_hbm, dst_hbm, zero_hbm, part_hbm, sidx, didx, *rest):
    rows = rest[:K]
    acc = rest[K]
    sems = rest[K + 1:]
    c = lax.axis_index("c")
    s = lax.axis_index("s")
    wid = s * NC + c
    r0 = s * RPT
    pltpu.sync_copy(zero_hbm.at[pl.ds(r0, RPT)], acc.at[pl.ds(r0, RPT)])
    plsc.subcore_barrier()

    def burst(cbase, j0, nb):
        offs = [pl.multiple_of(cbase + (j0 + b) * CH, CH) for b in range(nb)]
        ds = []
        for b in range(nb):
            pltpu.sync_copy(src_hbm.at[pl.ds(offs[b], CH)], sidx.at[b])
            ds.append(pltpu.async_copy(hs_hbm.at[sidx.at[b]], rows[b], sems[b]))
        for b in range(nb):
            pltpu.sync_copy(dst_hbm.at[pl.ds(offs[b], CH)], didx.at[b])
        for b in range(nb):
            ds[b].wait()
            pltpu.sync_copy(rows[b], acc.at[didx.at[b]], add=True)

    def run(nch, cbase):
        nt = nch // K
        for b in range(K):
            off = pl.multiple_of(cbase + b * CH, CH)
            pltpu.sync_copy(src_hbm.at[pl.ds(off, CH)], sidx.at[b])
            pltpu.async_copy(hs_hbm.at[sidx.at[b]], rows[b], sems[b])
            pltpu.sync_copy(dst_hbm.at[pl.ds(off, CH)], didx.at[b])

        def step(jj, carry):
            for b in range(K):
                pltpu.make_async_copy(hs_hbm.at[sidx.at[b]], rows[b], sems[b]).wait()
                pltpu.sync_copy(rows[b], acc.at[didx.at[b]], add=True)
                offn = pl.multiple_of(cbase + ((jj + 1) * K + b) * CH, CH)
                pltpu.sync_copy(src_hbm.at[pl.ds(offn, CH)], sidx.at[b])
                pltpu.async_copy(hs_hbm.at[sidx.at[b]], rows[b], sems[b])
                pltpu.sync_copy(dst_hbm.at[pl.ds(offn, CH)], didx.at[b])
            return carry

        lax.fori_loop(0, nt - 1, step, 0)
        for b in range(K):
            pltpu.make_async_copy(hs_hbm.at[sidx.at[b]], rows[b], sems[b]).wait()
            pltpu.sync_copy(rows[b], acc.at[didx.at[b]], add=True)
        if nch % K:
            burst(cbase, nt * K, nch % K)

    @pl.when(c == 0)
    def _():
        run(N0, s * (N0 * CH))

    @pl.when(c == 1)
    def _():
        run(N1, NS * N0 * CH + s * (N1 * CH))

    plsc.subcore_barrier()
    pltpu.sync_copy(acc.at[pl.ds(r0, RPT)], part_hbm.at[c, pl.ds(r0, RPT)])


def _tc_dis_body(degpart_ref, dis_ref):
    d = jnp.sum(degpart_ref[...], axis=0, keepdims=True)
    dis_ref[...] = jnp.where(d > 0, lax.rsqrt(jnp.maximum(d, 1e-12)), 0.0)


_tc_dis = pl.pallas_call(
    _tc_dis_body,
    out_shape=jax.ShapeDtypeStruct((1, NHIST), jnp.float32),
)

BLK = 1000
GRID = N // BLK


def _tc_l1_body(x_ref, dis_ref, w_ref, v_ref, b_ref, hs_ref, xvb_ref):
    xb = x_ref[...]
    dis = dis_ref[...]
    hs_ref[...] = jnp.dot(xb, w_ref[...], preferred_element_type=jnp.float32) * dis
    xvb_ref[...] = jnp.dot(xb, v_ref[...], preferred_element_type=jnp.float32) + b_ref[...]


_tc_l1 = pl.pallas_call(
    _tc_l1_body,
    grid=(GRID,),
    in_specs=[
        pl.BlockSpec((BLK, D), lambda k: (k, 0)),
        pl.BlockSpec((BLK, 1), lambda k: (k, 0)),
        pl.BlockSpec((D, D), lambda k: (0, 0)),
        pl.BlockSpec((D, D), lambda k: (0, 0)),
        pl.BlockSpec((1, D), lambda k: (0, 0)),
    ],
    out_specs=[
        pl.BlockSpec((BLK, D), lambda k: (k, 0)),
        pl.BlockSpec((BLK, D), lambda k: (k, 0)),
    ],
    out_shape=[
        jax.ShapeDtypeStruct((N, D), jnp.float32),
        jax.ShapeDtypeStruct((N, D), jnp.float32),
    ],
)


def _tc_mid_body(p_ref, xvb_ref, dis_ref, w_ref, v_ref, b_ref, hs_ref, xvb2_ref):
    dis = dis_ref[...]
    h = jax.nn.relu(dis * (p_ref[0] + p_ref[1]) + xvb_ref[...])
    hs_ref[...] = jnp.dot(h, w_ref[...], preferred_element_type=jnp.float32) * dis
    xvb2_ref[...] = jnp.dot(h, v_ref[...], preferred_element_type=jnp.float32) + b_ref[...]


_tc_mid = pl.pallas_call(
    _tc_mid_body,
    grid=(GRID,),
    in_specs=[
        pl.BlockSpec((NC, BLK, D), lambda k: (0, k, 0)),
        pl.BlockSpec((BLK, D), lambda k: (k, 0)),
        pl.BlockSpec((BLK, 1), lambda k: (k, 0)),
        pl.BlockSpec((D, D), lambda k: (0, 0)),
        pl.BlockSpec((D, D), lambda k: (0, 0)),
        pl.BlockSpec((1, D), lambda k: (0, 0)),
    ],
    out_specs=[
        pl.BlockSpec((BLK, D), lambda k: (k, 0)),
        pl.BlockSpec((BLK, D), lambda k: (k, 0)),
    ],
    out_shape=[
        jax.ShapeDtypeStruct((N, D), jnp.float32),
        jax.ShapeDtypeStruct((N, D), jnp.float32),
    ],
)


def _tc_pool_body(p_ref, xvb_ref, dis_ref, batch_ref, wfc_ref, bfc_ref, out_ref, sums, cnt):
    k = pl.program_id(0)

    @pl.when(k == 0)
    def _():
        sums[...] = jnp.zeros((G, D), jnp.float32)
        cnt[...] = jnp.zeros((G, D), jnp.float32)

    dis = dis_ref[...]
    h = jax.nn.relu(dis * (p_ref[0] + p_ref[1]) + xvb_ref[...])
    oh = (batch_ref[...] == lax.broadcasted_iota(jnp.int32, (BLK, G), 1)).astype(jnp.float32)
    sums[...] += lax.dot_general(oh, h, (((0,), (0,)), ((), ())),
                                 preferred_element_type=jnp.float32)
    cnt[...] += lax.dot_general(oh, jnp.ones((BLK, D), jnp.float32),
                                (((0,), (0,)), ((), ())),
                                preferred_element_type=jnp.float32)

    @pl.when(k == GRID - 1)
    def _():
        pooled = sums[...] / jnp.maximum(cnt[...], 1.0)
        out_ref[...] = jnp.dot(pooled, wfc_ref[...],
                               preferred_element_type=jnp.float32) + bfc_ref[...]


_tc_pool = pl.pallas_call(
    _tc_pool_body,
    grid=(GRID,),
    in_specs=[
        pl.BlockSpec((NC, BLK, D), lambda k: (0, k, 0)),
        pl.BlockSpec((BLK, D), lambda k: (k, 0)),
        pl.BlockSpec((BLK, 1), lambda k: (k, 0)),
        pl.BlockSpec((BLK, 1), lambda k: (k, 0)),
        pl.BlockSpec((D, DOUT), lambda k: (0, 0)),
        pl.BlockSpec((1, DOUT), lambda k: (0, 0)),
    ],
    out_specs=pl.BlockSpec((G, DOUT), lambda k: (0, 0)),
    out_shape=jax.ShapeDtypeStruct((G, DOUT), jnp.float32),
    scratch_shapes=[
        pltpu.VMEM((G, D), jnp.float32),
        pltpu.VMEM((G, D), jnp.float32),
    ],
)


def kernel(x, edge_index, batch, W1, V1, b1, W2, V2, b2, Wfc, bfc):
    src = edge_index[0]
    dst = edge_index[1]
    pad = EPAD - E
    srcp = jnp.concatenate([src, jnp.zeros((pad,), jnp.int32)])
    dstp = jnp.concatenate([dst, jnp.full((pad,), N, jnp.int32)])
    zero = jnp.zeros((NROWPAD, D), jnp.float32)
    zero1 = jnp.zeros((NHIST,), jnp.float32)

    degflat = _sc_degree(dstp, zero1)
    dis = _tc_dis(degflat.reshape(NW, NHIST))
    dis_col = dis.reshape(-1)[:N].reshape(N, 1)

    hs1, xvb1 = _tc_l1(x, dis_col, W1, V1, b1.reshape(1, D))
    part1 = _sc_scatter(hs1, srcp, dstp, zero)
    hs2, xvb2 = _tc_mid(part1, xvb1, dis_col, W2, V2, b2.reshape(1, D))
    part2 = _sc_scatter(hs2, srcp, dstp, zero)
    return _tc_pool(part2, xvb2, dis_col, batch.reshape(N, 1), Wfc, bfc.reshape(1, DOUT))


# speedup vs baseline: 1.0050x; 1.0050x over previous
import functools

import jax
import jax.numpy as jnp
from jax import lax
from jax.experimental import pallas as pl
from jax.experimental.pallas import tpu as pltpu
from jax.experimental.pallas import tpu_sc as plsc

N = 10000
E = 320000
D = 128
G = 16
DOUT = 64

NC = 2
NS = 16
L = 16
NW = NC * NS

CH = 128
NCHUNK = 79
N0 = 120
N1 = 38
EPT = NCHUNK * CH
EPAD = EPT * NW
NROWPAD = 10112
RPT = NROWPAD // NS
HR = EPT // L
DEGR = 80
NHIST = DEGR * D

_sc_mesh = plsc.VectorSubcoreMesh(core_axis_name="c", subcore_axis_name="s")
_sc_params = pltpu.CompilerParams(needs_layout_passes=False)


@functools.partial(
    pl.kernel,
    out_type=jax.ShapeDtypeStruct((NW * NHIST,), jnp.float32),
    mesh=_sc_mesh,
    compiler_params=_sc_params,
    scratch_types=[
        pltpu.VMEM((EPT,), jnp.int32),
        pltpu.VMEM((NHIST,), jnp.float32),
    ],
)
def _sc_degree(dst_hbm, zero_hbm, deg_hbm, didx, hist):
    c = lax.axis_index("c")
    s = lax.axis_index("s")
    wid = s * NC + c
    base = pl.multiple_of(wid * EPT, CH)
    pltpu.sync_copy(dst_hbm.at[pl.ds(base, EPT)], didx)
    pltpu.sync_copy(zero_hbm, hist)
    ones = jnp.ones((L,), jnp.float32)

    def step(j, carry):
        idx = didx[pl.ds(j * L, L)]
        plsc.addupdate_scatter(hist, [idx], ones)
        return carry

    lax.fori_loop(0, HR, step, 0)
    pltpu.sync_copy(hist, deg_hbm.at[pl.ds(pl.multiple_of(wid * NHIST, CH), NHIST)])


K = 3


@functools.partial(
    pl.kernel,
    out_type=jax.ShapeDtypeStruct((NC, NROWPAD, D), jnp.float32),
    mesh=_sc_mesh,
    compiler_params=_sc_params,
    scratch_types=[
        pltpu.VMEM((K, CH), jnp.int32),
        pltpu.VMEM((K, CH), jnp.int32),
    ] + [pltpu.VMEM((CH, D), jnp.float32) for _ in range(K)] + [
        pltpu.VMEM_SHARED((NROWPAD, D), jnp.float32),
    ] + [pltpu.SemaphoreType.DMA for _ in range(K)],
)
def _sc_scatter(hs_hbm, src_hbm, dst_hbm, zero_hbm, part_hbm, sidx, didx, *rest):
    rows = rest[:K]
    acc = rest[K]
    sems = rest[K + 1:]
    c = lax.axis_index("c")
    s = lax.axis_index("s")
    wid = s * NC + c
    r0 = s * RPT
    pltpu.sync_copy(zero_hbm.at[pl.ds(r0, RPT)], acc.at[pl.ds(r0, RPT)])
    plsc.subcore_barrier()

    def burst(cbase, j0, nb):
        offs = [pl.multiple_of(cbase + (j0 + b) * CH, CH) for b in range(nb)]
        ds = []
        for b in range(nb):
            pltpu.sync_copy(src_hbm.at[pl.ds(offs[b], CH)], sidx.at[b])
            ds.append(pltpu.async_copy(hs_hbm.at[sidx.at[b]], rows[b], sems[b]))
        for b in range(nb):
            pltpu.sync_copy(dst_hbm.at[pl.ds(offs[b], CH)], didx.at[b])
        for b in range(nb):
            ds[b].wait()
            pltpu.sync_copy(rows[b], acc.at[didx.at[b]], add=True)

    def run(nch, cbase):
        nt = nch // K
        for b in range(K):
            off = pl.multiple_of(cbase + b * CH, CH)
            pltpu.sync_copy(src_hbm.at[pl.ds(off, CH)], sidx.at[b])
            pltpu.async_copy(hs_hbm.at[sidx.at[b]], rows[b], sems[b])
            pltpu.sync_copy(dst_hbm.at[pl.ds(off, CH)], didx.at[b])

        def step(jj, carry):
            for b in range(K):
                pltpu.make_async_copy(hs_hbm.at[sidx.at[b]], rows[b], sems[b]).wait()
                pltpu.sync_copy(rows[b], acc.at[didx.at[b]], add=True)
                offn = pl.multiple_of(cbase + ((jj + 1) * K + b) * CH, CH)
                pltpu.sync_copy(src_hbm.at[pl.ds(offn, CH)], sidx.at[b])
                pltpu.async_copy(hs_hbm.at[sidx.at[b]], rows[b], sems[b])
                pltpu.sync_copy(dst_hbm.at[pl.ds(offn, CH)], didx.at[b])
            return carry

        lax.fori_loop(0, nt - 1, step, 0)
        for b in range(K):
            pltpu.make_async_copy(hs_hbm.at[sidx.at[b]], rows[b], sems[b]).wait()
            pltpu.sync_copy(rows[b], acc.at[didx.at[b]], add=True)
        if nch % K:
            burst(cbase, nt * K, nch % K)

    @pl.when(c == 0)
    def _():
        run(N0, s * (N0 * CH))

    @pl.when(c == 1)
    def _():
        run(N1, NS * N0 * CH + s * (N1 * CH))

    plsc.subcore_barrier()
    pltpu.sync_copy(acc.at[pl.ds(r0, RPT)], part_hbm.at[c, pl.ds(r0, RPT)])


def _tc_dis_body(degpart_ref, dis_ref):
    d = jnp.sum(degpart_ref[...], axis=0, keepdims=True)
    dis_ref[...] = jnp.where(d > 0, lax.rsqrt(jnp.maximum(d, 1e-12)), 0.0)


_tc_dis = pl.pallas_call(
    _tc_dis_body,
    out_shape=jax.ShapeDtypeStruct((1, NHIST), jnp.float32),
)

BLK = 1000
GRID = N // BLK


def _tc_l1_body(x_ref, dis_ref, w_ref, v_ref, b_ref, hs_ref, xvb_ref):
    xb = x_ref[...]
    dis = dis_ref[...]
    hs_ref[...] = jnp.dot(xb, w_ref[...], preferred_element_type=jnp.float32) * dis
    xvb_ref[...] = jnp.dot(xb, v_ref[...], preferred_element_type=jnp.float32) + b_ref[...]


_tc_l1 = pl.pallas_call(
    _tc_l1_body,
    grid=(GRID,),
    in_specs=[
        pl.BlockSpec((BLK, D), lambda k: (k, 0)),
        pl.BlockSpec((BLK, 1), lambda k: (k, 0)),
        pl.BlockSpec((D, D), lambda k: (0, 0)),
        pl.BlockSpec((D, D), lambda k: (0, 0)),
        pl.BlockSpec((1, D), lambda k: (0, 0)),
    ],
    out_specs=[
        pl.BlockSpec((BLK, D), lambda k: (k, 0)),
        pl.BlockSpec((BLK, D), lambda k: (k, 0)),
    ],
    out_shape=[
        jax.ShapeDtypeStruct((N, D), jnp.float32),
        jax.ShapeDtypeStruct((N, D), jnp.float32),
    ],
)


def _tc_mid_body(p_ref, xvb_ref, dis_ref, w_ref, v_ref, b_ref, hs_ref, xvb2_ref):
    dis = dis_ref[...]
    h = jax.nn.relu(dis * (p_ref[0] + p_ref[1]) + xvb_ref[...])
    hs_ref[...] = jnp.dot(h, w_ref[...], preferred_element_type=jnp.float32) * dis
    xvb2_ref[...] = jnp.dot(h, v_ref[...], preferred_element_type=jnp.float32) + b_ref[...]


_tc_mid = pl.pallas_call(
    _tc_mid_body,
    grid=(GRID,),
    in_specs=[
        pl.BlockSpec((NC, BLK, D), lambda k: (0, k, 0)),
        pl.BlockSpec((BLK, D), lambda k: (k, 0)),
        pl.BlockSpec((BLK, 1), lambda k: (k, 0)),
        pl.BlockSpec((D, D), lambda k: (0, 0)),
        pl.BlockSpec((D, D), lambda k: (0, 0)),
        pl.BlockSpec((1, D), lambda k: (0, 0)),
    ],
    out_specs=[
        pl.BlockSpec((BLK, D), lambda k: (k, 0)),
        pl.BlockSpec((BLK, D), lambda k: (k, 0)),
    ],
    out_shape=[
        jax.ShapeDtypeStruct((N, D), jnp.float32),
        jax.ShapeDtypeStruct((N, D), jnp.float32),
    ],
)


def _tc_pool_body(p_ref, xvb_ref, dis_ref, batch_ref, wfc_ref, bfc_ref, out_ref, sums, cnt):
    k = pl.program_id(0)

    @pl.when(k == 0)
    def _():
        sums[...] = jnp.zeros((G, D), jnp.float32)
        cnt[...] = jnp.zeros((G, D), jnp.float32)

    dis = dis_ref[...]
    h = jax.nn.relu(dis * (p_ref[0] + p_ref[1]) + xvb_ref[...])
    oh = (batch_ref[...] == lax.broadcasted_iota(jnp.int32, (BLK, G), 1)).astype(jnp.float32)
    sums[...] += lax.dot_general(oh, h, (((0,), (0,)), ((), ())),
                                 preferred_element_type=jnp.float32)
    cnt[...] += lax.dot_general(oh, jnp.ones((BLK, D), jnp.float32),
                                (((0,), (0,)), ((), ())),
                                preferred_element_type=jnp.float32)

    @pl.when(k == GRID - 1)
    def _():
        pooled = sums[...] / jnp.maximum(cnt[...], 1.0)
        out_ref[...] = jnp.dot(pooled, wfc_ref[...],
                               preferred_element_type=jnp.float32) + bfc_ref[...]


_tc_pool = pl.pallas_call(
    _tc_pool_body,
    grid=(GRID,),
    in_specs=[
        pl.BlockSpec((NC, BLK, D), lambda k: (0, k, 0)),
        pl.BlockSpec((BLK, D), lambda k: (k, 0)),
        pl.BlockSpec((BLK, 1), lambda k: (k, 0)),
        pl.BlockSpec((BLK, 1), lambda k: (k, 0)),
        pl.BlockSpec((D, DOUT), lambda k: (0, 0)),
        pl.BlockSpec((1, DOUT), lambda k: (0, 0)),
    ],
    out_specs=pl.BlockSpec((G, DOUT), lambda k: (0, 0)),
    out_shape=jax.ShapeDtypeStruct((G, DOUT), jnp.float32),
    scratch_shapes=[
        pltpu.VMEM((G, D), jnp.float32),
        pltpu.VMEM((G, D), jnp.float32),
    ],
)


def kernel(x, edge_index, batch, W1, V1, b1, W2, V2, b2, Wfc, bfc):
    src = edge_index[0]
    dst = edge_index[1]
    pad = EPAD - E
    srcp = jnp.concatenate([src, jnp.zeros((pad,), jnp.int32)])
    dstp = jnp.concatenate([dst, jnp.full((pad,), N, jnp.int32)])
    zero = jnp.zeros((NROWPAD, D), jnp.float32)
    zero1 = jnp.zeros((NHIST,), jnp.float32)

    degflat = _sc_degree(dstp, zero1)
    dis = _tc_dis(degflat.reshape(NW, NHIST))
    dis_col = dis.reshape(-1)[:N].reshape(N, 1)

    hs1, xvb1 = _tc_l1(x, dis_col, W1, V1, b1.reshape(1, D))
    part1 = _sc_scatter(hs1, srcp, dstp, zero)
    hs2, xvb2 = _tc_mid(part1, xvb1, dis_col, W2, V2, b2.reshape(1, D))
    part2 = _sc_scatter(hs2, srcp, dstp, zero)
    return _tc_pool(part2, xvb2, dis_col, batch.reshape(N, 1), Wfc, bfc.reshape(1, DOUT))
